# baseline (speedup 1.0000x reference)
import jax
import jax.numpy as jnp
from jax.experimental import pallas as pl
from jax.experimental.pallas import tpu as pltpu

_HIDDEN = 2048
_N_EXPERTS = 64
_TOP_K = 8
_TB = 2048


def _router_body(x_ref, w_ref, b_ref, idx_ref, wgt_ref):
    x = x_ref[...]
    w = w_ref[...]
    logits = jax.lax.dot_general(
        w, x, (((1,), (1,)), ((), ())), preferred_element_type=jnp.float32
    )
    scores = jax.nn.sigmoid(logits)
    sel = scores + b_ref[...]

    row = jax.lax.broadcasted_iota(jnp.int32, (_N_EXPERTS, _TB), 0).astype(
        jnp.float32
    )
    row8 = jax.lax.broadcasted_iota(jnp.int32, (_TOP_K, _TB), 0)
    idx_acc = jnp.zeros((_TOP_K, _TB), jnp.float32)
    wgt_acc = jnp.zeros((_TOP_K, _TB), jnp.float32)
    neg_inf = jnp.float32(-jnp.inf)

    for k in range(_TOP_K):
        m = jnp.max(sel, axis=0, keepdims=True)
        is_max = sel == m
        idx = jnp.min(
            jnp.where(is_max, row, float(_N_EXPERTS)), axis=0, keepdims=True
        )
        onehot = row == idx
        wk = jnp.sum(jnp.where(onehot, scores, 0.0), axis=0, keepdims=True)
        idx_acc = idx_acc + jnp.where(row8 == k, idx, 0.0)
        wgt_acc = wgt_acc + jnp.where(row8 == k, wk, 0.0)
        sel = jnp.where(onehot, neg_inf, sel)

    denom = jnp.sum(wgt_acc, axis=0, keepdims=True) + 1e-20
    idx_ref[...] = idx_acc.astype(jnp.int32).T
    wgt_ref[...] = (wgt_acc / denom).T


@jax.jit
def kernel(hidden_states, weight, e_score_correction_bias):
    batch, seq, hidden = hidden_states.shape
    tokens = batch * seq
    x = hidden_states.reshape(tokens, hidden)
    bias2d = e_score_correction_bias.reshape(_N_EXPERTS, 1)
    grid = (tokens // _TB,)
    idx_t, wgt_t = pl.pallas_call(
        _router_body,
        grid=grid,
        in_specs=[
            pl.BlockSpec((_TB, hidden), lambda i: (i, 0)),
            pl.BlockSpec((_N_EXPERTS, hidden), lambda i: (0, 0)),
            pl.BlockSpec((_N_EXPERTS, 1), lambda i: (0, 0)),
        ],
        out_specs=[
            pl.BlockSpec((_TB, _TOP_K), lambda i: (i, 0)),
            pl.BlockSpec((_TB, _TOP_K), lambda i: (i, 0)),
        ],
        out_shape=[
            jax.ShapeDtypeStruct((tokens, _TOP_K), jnp.int32),
            jax.ShapeDtypeStruct((tokens, _TOP_K), jnp.float32),
        ],
        compiler_params=pltpu.CompilerParams(
            dimension_semantics=("arbitrary",),
        ),
    )(x, weight, bias2d)
    return idx_t, wgt_t


# speedup vs baseline: 1.0953x; 1.0953x over previous
import jax
import jax.numpy as jnp
from jax.experimental import pallas as pl
from jax.experimental.pallas import tpu as pltpu

_HIDDEN = 2048
_N_EXPERTS = 64
_TOP_K = 8
_TB = 4096
_HC = 1024
_NH = _HIDDEN // _HC


def _router_body(x_ref, w_ref, b_ref, idx_ref, wgt_ref, acc_ref):
    h = pl.program_id(1)
    x = x_ref[...]
    w = w_ref[...]
    part = jax.lax.dot_general(
        w, x, (((1,), (1,)), ((), ())), preferred_element_type=jnp.float32
    )

    @pl.when(h == 0)
    def _init():
        acc_ref[...] = part

    @pl.when(h != 0)
    def _acc():
        acc_ref[...] += part

    @pl.when(h == _NH - 1)
    def _finish():
        logits = acc_ref[...]
        scores = jax.nn.sigmoid(logits)
        sel = scores + b_ref[...]

        row = jax.lax.broadcasted_iota(
            jnp.int32, (_N_EXPERTS, _TB), 0
        ).astype(jnp.float32)
        row8 = jax.lax.broadcasted_iota(jnp.int32, (_TOP_K, _TB), 0)
        idx_acc = jnp.zeros((_TOP_K, _TB), jnp.float32)
        wgt_acc = jnp.zeros((_TOP_K, _TB), jnp.float32)
        neg_inf = jnp.float32(-jnp.inf)

        for k in range(_TOP_K):
            m = jnp.max(sel, axis=0, keepdims=True)
            is_max = sel == m
            idx = jnp.min(
                jnp.where(is_max, row, float(_N_EXPERTS)),
                axis=0,
                keepdims=True,
            )
            onehot = row == idx
            wk = jnp.sum(
                jnp.where(onehot, scores, 0.0), axis=0, keepdims=True
            )
            idx_acc = idx_acc + jnp.where(row8 == k, idx, 0.0)
            wgt_acc = wgt_acc + jnp.where(row8 == k, wk, 0.0)
            sel = jnp.where(onehot, neg_inf, sel)

        denom = jnp.sum(wgt_acc, axis=0, keepdims=True) + 1e-20
        idx_ref[...] = idx_acc.astype(jnp.int32)
        wgt_ref[...] = wgt_acc / denom


@jax.jit
def kernel(hidden_states, weight, e_score_correction_bias):
    batch, seq, hidden = hidden_states.shape
    tokens = batch * seq
    x = hidden_states.reshape(tokens, hidden)
    bias2d = e_score_correction_bias.reshape(_N_EXPERTS, 1)
    grid = (tokens // _TB, _NH)
    idx_t, wgt_t = pl.pallas_call(
        _router_body,
        grid=grid,
        in_specs=[
            pl.BlockSpec((_TB, _HC), lambda i, h: (i, h)),
            pl.BlockSpec((_N_EXPERTS, _HC), lambda i, h: (0, h)),
            pl.BlockSpec((_N_EXPERTS, 1), lambda i, h: (0, 0)),
        ],
        out_specs=[
            pl.BlockSpec((_TOP_K, _TB), lambda i, h: (0, i)),
            pl.BlockSpec((_TOP_K, _TB), lambda i, h: (0, i)),
        ],
        out_shape=[
            jax.ShapeDtypeStruct((_TOP_K, tokens), jnp.int32),
            jax.ShapeDtypeStruct((_TOP_K, tokens), jnp.float32),
        ],
        scratch_shapes=[pltpu.VMEM((_N_EXPERTS, _TB), jnp.float32)],
        compiler_params=pltpu.CompilerParams(
            dimension_semantics=("arbitrary", "arbitrary"),
        ),
    )(x, weight, bias2d)
    return idx_t.T, wgt_t.T


# speedup vs baseline: 1.3308x; 1.2151x over previous
import jax
import jax.numpy as jnp
from jax.experimental import pallas as pl
from jax.experimental.pallas import tpu as pltpu

_HIDDEN = 2048
_N_EXPERTS = 64
_TOP_K = 8
_TB = 2048


def _router_body(x_ref, w_ref, b_ref, idx_ref, wgt_ref):
    x = x_ref[...]
    w = w_ref[...]
    logits = jax.lax.dot_general(
        w, x, (((1,), (1,)), ((), ())), preferred_element_type=jnp.float32
    )
    scores = jax.nn.sigmoid(logits)
    sel = scores + b_ref[...]

    row = jax.lax.broadcasted_iota(jnp.int32, (_N_EXPERTS, _TB), 0).astype(
        jnp.float32
    )
    row8 = jax.lax.broadcasted_iota(jnp.int32, (_TOP_K, _TB), 0)
    idx_acc = jnp.zeros((_TOP_K, _TB), jnp.float32)
    wgt_acc = jnp.zeros((_TOP_K, _TB), jnp.float32)
    neg_inf = jnp.float32(-jnp.inf)

    for k in range(_TOP_K):
        m = jnp.max(sel, axis=0, keepdims=True)
        is_max = sel == m
        idx = jnp.min(
            jnp.where(is_max, row, float(_N_EXPERTS)), axis=0, keepdims=True
        )
        onehot = row == idx
        wk = jnp.sum(jnp.where(onehot, scores, 0.0), axis=0, keepdims=True)
        idx_acc = idx_acc + jnp.where(row8 == k, idx, 0.0)
        wgt_acc = wgt_acc + jnp.where(row8 == k, wk, 0.0)
        sel = jnp.where(onehot, neg_inf, sel)

    denom = jnp.sum(wgt_acc, axis=0, keepdims=True) + 1e-20
    idx_ref[...] = idx_acc.astype(jnp.int32)
    wgt_ref[...] = wgt_acc / denom


@jax.jit
def kernel(hidden_states, weight, e_score_correction_bias):
    batch, seq, hidden = hidden_states.shape
    tokens = batch * seq
    x = hidden_states.reshape(tokens, hidden)
    bias2d = e_score_correction_bias.reshape(_N_EXPERTS, 1)
    grid = (tokens // _TB,)
    idx_t, wgt_t = pl.pallas_call(
        _router_body,
        grid=grid,
        in_specs=[
            pl.BlockSpec((_TB, hidden), lambda i: (i, 0)),
            pl.BlockSpec((_N_EXPERTS, hidden), lambda i: (0, 0)),
            pl.BlockSpec((_N_EXPERTS, 1), lambda i: (0, 0)),
        ],
        out_specs=[
            pl.BlockSpec((_TOP_K, _TB), lambda i: (0, i)),
            pl.BlockSpec((_TOP_K, _TB), lambda i: (0, i)),
        ],
        out_shape=[
            jax.ShapeDtypeStruct((_TOP_K, tokens), jnp.int32),
            jax.ShapeDtypeStruct((_TOP_K, tokens), jnp.float32),
        ],
        compiler_params=pltpu.CompilerParams(
            dimension_semantics=("parallel",),
        ),
    )(x, weight, bias2d)
    return idx_t.T, wgt_t.T


# speedup vs baseline: 1.3777x; 1.0352x over previous
import jax
import jax.numpy as jnp
from jax.experimental import pallas as pl
from jax.experimental.pallas import tpu as pltpu

_HIDDEN = 2048
_N_EXPERTS = 64
_TOP_K = 8
_TB = 2048


def _router_body(x_ref, w_ref, b_ref, idx_ref, wgt_ref):
    x = x_ref[...]
    w = w_ref[...]
    logits = jax.lax.dot_general(
        w, x, (((1,), (1,)), ((), ())), preferred_element_type=jnp.float32
    )
    scores = jax.nn.sigmoid(logits)
    sel = scores + b_ref[...]

    row = jax.lax.broadcasted_iota(jnp.int32, (_N_EXPERTS, _TB), 0).astype(
        jnp.float32
    )
    row8 = jax.lax.broadcasted_iota(jnp.int32, (_TOP_K, _TB), 0)
    idx_acc = jnp.zeros((_TOP_K, _TB), jnp.float32)
    wgt_acc = jnp.zeros((_TOP_K, _TB), jnp.float32)
    neg_inf = jnp.float32(-jnp.inf)

    for k in range(_TOP_K):
        m = jnp.max(sel, axis=0, keepdims=True)
        is_max = sel == m
        idx = jnp.min(
            jnp.where(is_max, row, float(_N_EXPERTS)), axis=0, keepdims=True
        )
        onehot = row == idx
        wk = jnp.sum(jnp.where(onehot, scores, 0.0), axis=0, keepdims=True)
        idx_acc = idx_acc + jnp.where(row8 == k, idx, 0.0)
        wgt_acc = wgt_acc + jnp.where(row8 == k, wk, 0.0)
        sel = jnp.where(onehot, neg_inf, sel)

    denom = jnp.sum(wgt_acc, axis=0, keepdims=True) + 1e-20
    idx_ref[...] = idx_acc.astype(jnp.int32)
    wgt_ref[...] = wgt_acc / denom


@jax.jit
def kernel(hidden_states, weight, e_score_correction_bias):
    batch, seq, hidden = hidden_states.shape
    tokens = batch * seq
    x = hidden_states.reshape(tokens, hidden)
    bias2d = e_score_correction_bias.reshape(_N_EXPERTS, 1)
    grid = (tokens // _TB,)
    idx_t, wgt_t = pl.pallas_call(
        _router_body,
        grid=grid,
        in_specs=[
            pl.BlockSpec((_TB, hidden), lambda i: (i, 0)),
            pl.BlockSpec((_N_EXPERTS, hidden), lambda i: (0, 0)),
            pl.BlockSpec((_N_EXPERTS, 1), lambda i: (0, 0)),
        ],
        out_specs=[
            pl.BlockSpec((_TOP_K, _TB), lambda i: (0, i)),
            pl.BlockSpec((_TOP_K, _TB), lambda i: (0, i)),
        ],
        out_shape=[
            jax.ShapeDtypeStruct((_TOP_K, tokens), jnp.int32),
            jax.ShapeDtypeStruct((_TOP_K, tokens), jnp.float32),
        ],
        compiler_params=pltpu.CompilerParams(
            dimension_semantics=("arbitrary",),
        ),
    )(x, weight, bias2d)
    return idx_t.T, wgt_t.T
